# baseline (speedup 1.0000x reference)
import functools

import jax
import jax.numpy as jnp
from jax import lax
from jax.experimental import pallas as pl
from jax.experimental.pallas import tpu as pltpu
from jax.experimental.pallas import tpu_sc as plsc

NC = 2
NS = 16
LN = 16
RPT = 640
EXCH = 512
SCH = 640
WK = 32
WCAP = 11264
WSZ = WCAP + 224


def _tc_pre(x, W, a_s, a_d, C, P):
    N = x.shape[0]
    F = x.shape[1]
    BN = 1000

    def body(x_ref, w_ref, as_ref, ad_ref, hx_ref, aa_ref):
        h = jnp.dot(x_ref[...], w_ref[...], preferred_element_type=jnp.float32)
        hx_ref[:, :C] = h
        col = lax.broadcasted_iota(jnp.int32, (BN, P), 1)
        hx_ref[:, C:] = jnp.where(col == 0, 1.0, 0.0)
        a1 = jnp.sum(h * as_ref[...], axis=1)
        a2 = jnp.sum(h * ad_ref[...], axis=1)
        aa_ref[...] = jnp.stack([a1, a2], axis=1)

    return pl.pallas_call(
        body,
        grid=(N // BN,),
        in_specs=[
            pl.BlockSpec((BN, F), lambda i: (i, 0)),
            pl.BlockSpec((F, C), lambda i: (0, 0)),
            pl.BlockSpec((1, C), lambda i: (0, 0)),
            pl.BlockSpec((1, C), lambda i: (0, 0)),
        ],
        out_specs=[
            pl.BlockSpec((BN, C + P), lambda i: (i, 0)),
            pl.BlockSpec((BN, 2), lambda i: (i, 0)),
        ],
        out_shape=[
            jax.ShapeDtypeStruct((N, C + P), jnp.float32),
            jax.ShapeDtypeStruct((N, 2), jnp.float32),
        ],
    )(x, W, a_s.reshape(1, C), a_d.reshape(1, C))


def _tc_mid(U, b, W, a_s, a_d, C, P, C2, P2):
    N = U.shape[1]
    BN = 1000

    def body(u_ref, b_ref, w_ref, as_ref, ad_ref, hx_ref, aa_ref):
        u = u_ref[0] + u_ref[1]
        o = u[:, :C] / (u[:, C:C + 1] + 1e-16) + b_ref[...]
        x2 = jnp.where(o > 0, o, jnp.exp(jnp.minimum(o, 0.0)) - 1.0)
        h = jnp.dot(x2, w_ref[...], preferred_element_type=jnp.float32)
        hx_ref[:, :C2] = h
        col = lax.broadcasted_iota(jnp.int32, (BN, P2), 1)
        hx_ref[:, C2:] = jnp.where(col == 0, 1.0, 0.0)
        a1 = jnp.sum(h * as_ref[...], axis=1)
        a2 = jnp.sum(h * ad_ref[...], axis=1)
        aa_ref[...] = jnp.stack([a1, a2], axis=1)

    return pl.pallas_call(
        body,
        grid=(N // BN,),
        in_specs=[
            pl.BlockSpec((2, BN, C + P), lambda i: (0, i, 0)),
            pl.BlockSpec((1, C), lambda i: (0, 0)),
            pl.BlockSpec((C, C2), lambda i: (0, 0)),
            pl.BlockSpec((1, C2), lambda i: (0, 0)),
            pl.BlockSpec((1, C2), lambda i: (0, 0)),
        ],
        out_specs=[
            pl.BlockSpec((BN, C2 + P2), lambda i: (i, 0)),
            pl.BlockSpec((BN, 2), lambda i: (i, 0)),
        ],
        out_shape=[
            jax.ShapeDtypeStruct((N, C2 + P2), jnp.float32),
            jax.ShapeDtypeStruct((N, 2), jnp.float32),
        ],
    )(U, b.reshape(1, C), W, a_s.reshape(1, C2), a_d.reshape(1, C2))


def _tc_post(U, b, C, P):
    N = U.shape[1]
    BN = 1000

    def body(u_ref, b_ref, out_ref):
        u = u_ref[0] + u_ref[1]
        o = u[:, :C] / (u[:, C:C + 1] + 1e-16) + b_ref[...]
        m = jnp.max(o, axis=1, keepdims=True)
        s = o - m
        lse = jnp.log(jnp.sum(jnp.exp(s), axis=1, keepdims=True))
        out_ref[...] = s - lse

    return pl.pallas_call(
        body,
        grid=(N // BN,),
        in_specs=[
            pl.BlockSpec((2, BN, C + P), lambda i: (0, i, 0)),
            pl.BlockSpec((1, C), lambda i: (0, 0)),
        ],
        out_specs=pl.BlockSpec((BN, C), lambda i: (i, 0)),
        out_shape=jax.ShapeDtypeStruct((N, C), jnp.float32),
    )(U, b.reshape(1, C))


_SC_PARAMS = pltpu.CompilerParams(
    needs_layout_passes=False, use_tc_tiling_on_sc=False)


@functools.partial(jax.jit, static_argnames=("N", "E"))
def _sc_ex(aa, src_p, dst_p, N, E):
    EP = src_p.shape[0]
    EPT = EP // (NC * NS)
    mesh = plsc.VectorSubcoreMesh(core_axis_name="c", subcore_axis_name="s")

    @functools.partial(
        pl.kernel,
        out_type=jax.ShapeDtypeStruct((EP,), jnp.float32),
        mesh=mesh,
        compiler_params=_SC_PARAMS,
        scratch_types=[
            pltpu.VMEM((2 * N,), jnp.float32),
            pltpu.VMEM((EXCH,), jnp.int32),
            pltpu.VMEM((EXCH,), jnp.int32),
            pltpu.VMEM((EXCH,), jnp.float32),
        ],
    )
    def k(aa_hbm, src_hbm, dst_hbm, ex_hbm, aa_v, ss_v, sd_v, se_v):
        cid = lax.axis_index("c")
        sid = lax.axis_index("s")
        wid = cid * NS + sid
        pltpu.sync_copy(aa_hbm, aa_v)
        base = wid * EPT

        def chunk(c, carry):
            off = base + c * EXCH
            pltpu.sync_copy(src_hbm.at[pl.ds(off, EXCH)], ss_v)
            pltpu.sync_copy(dst_hbm.at[pl.ds(off, EXCH)], sd_v)
            for j in range(EXCH // LN):
                s16 = ss_v[pl.ds(j * LN, LN)]
                d16 = sd_v[pl.ds(j * LN, LN)]
                d16c = jnp.maximum(d16, 0)
                z = (plsc.load_gather(aa_v, [2 * s16])
                     + plsc.load_gather(aa_v, [2 * d16c + 1]))
                e = jnp.maximum(z, 0.2 * z)
                eid = off + j * LN + lax.iota(jnp.int32, LN)
                se_v[pl.ds(j * LN, LN)] = jnp.where(eid < E, jnp.exp(e), 0.0)
            pltpu.sync_copy(se_v, ex_hbm.at[pl.ds(off, EXCH)])
            return carry

        lax.fori_loop(0, EPT // EXCH, chunk, 0)

    return k(aa, src_p, dst_p)


@functools.partial(jax.jit, static_argnames=("CP",))
def _sc_acc(hx, ex, src_p, dst_p, CP):
    EP = src_p.shape[0]
    EH = EP // NC
    NV = CP // LN
    mesh = plsc.VectorSubcoreMesh(core_axis_name="c", subcore_axis_name="s")

    @functools.partial(
        pl.kernel,
        out_type=jax.ShapeDtypeStruct((NC, NS, RPT * CP), jnp.float32),
        mesh=mesh,
        compiler_params=_SC_PARAMS,
        scratch_types=[
            pltpu.VMEM((WSZ,), jnp.int32),
            pltpu.VMEM((WSZ,), jnp.float32),
            pltpu.VMEM((SCH,), jnp.int32),
            pltpu.VMEM((SCH,), jnp.int32),
            pltpu.VMEM((SCH,), jnp.float32),
            pltpu.VMEM((SCH,), jnp.int32),
            pltpu.VMEM((SCH,), jnp.int32),
            pltpu.VMEM((SCH,), jnp.float32),
            pltpu.VMEM((WK, CP), jnp.float32),
            pltpu.VMEM((WK, CP), jnp.float32),
            pltpu.VMEM((WK,), jnp.int32),
            pltpu.VMEM((WK,), jnp.int32),
            pltpu.VMEM((RPT * CP,), jnp.float32),
            pltpu.SemaphoreType.DMA,
            pltpu.SemaphoreType.DMA,
            pltpu.SemaphoreType.DMA,
            pltpu.SemaphoreType.DMA,
        ],
    )
    def k(hx_hbm, ex_hbm, src_hbm, dst_hbm, u_hbm,
          wlw_v, wle_v, ss0, sd0, se0, ss1, sd1, se1,
          rows0, rows1, sb0, sb1, acc_v, sem0, sem1, sems, semt):
        cid = lax.axis_index("c")
        sid = lax.axis_index("s")
        mybase = sid * RPT
        hb = cid * EH

        zero16 = jnp.zeros((LN,), jnp.float32)

        def zr(g, carry):
            for v in range(NV):
                acc_v[pl.ds(g * CP + v * LN, LN)] = zero16
            return carry

        lax.fori_loop(0, RPT, zr, 0)

        def scan_one(ss, sd, se, cur):
            for j in range(SCH // LN):
                s16 = ss[pl.ds(j * LN, LN)]
                d16 = sd[pl.ds(j * LN, LN)]
                e16 = se[pl.ds(j * LN, LN)]
                dl = d16 - mybase
                m = (dl >= 0) & (dl < RPT)
                w = s16 | (jnp.maximum(dl, 0) << 14)
                plsc.store_compressed(wlw_v.at[pl.ds(cur, LN)], w, mask=m)
                plsc.store_compressed(wle_v.at[pl.ds(cur, LN)], e16, mask=m)
                cur = cur + plsc.all_reduce_population_count(m)[0]
            return jnp.minimum(cur, WCAP)

        NSC = EH // SCH

        def issue_scan(c, ss, sd, se, sem):
            off = hb + jnp.minimum(c, NSC - 1) * SCH
            pltpu.async_copy(src_hbm.at[pl.ds(off, SCH)], ss, sem)
            pltpu.async_copy(dst_hbm.at[pl.ds(off, SCH)], sd, sem)
            pltpu.async_copy(ex_hbm.at[pl.ds(off, SCH)], se, sem)

        def wait_scan(ss, sd, se, sem):
            dummy = src_hbm.at[pl.ds(0, SCH)]
            pltpu.make_async_copy(dummy, ss, sem).wait()
            pltpu.make_async_copy(dummy, sd, sem).wait()
            pltpu.make_async_copy(ex_hbm.at[pl.ds(0, SCH)], se, sem).wait()

        issue_scan(0, ss0, sd0, se0, sems)

        def scan_pair(c2, cur):
            c0 = 2 * c2
            issue_scan(c0 + 1, ss1, sd1, se1, semt)
            wait_scan(ss0, sd0, se0, sems)
            cur = scan_one(ss0, sd0, se0, cur)
            issue_scan(c0 + 2, ss0, sd0, se0, sems)
            wait_scan(ss1, sd1, se1, semt)
            return scan_one(ss1, sd1, se1, cur)

        cur = lax.fori_loop(0, NSC // 2, scan_pair, 0)
        wait_scan(ss0, sd0, se0, sems)

        def ztail(t, carry):
            wlw_v[pl.ds(cur + t * LN, LN)] = jnp.zeros((LN,), jnp.int32)
            wle_v[pl.ds(cur + t * LN, LN)] = zero16
            return carry

        lax.fori_loop(0, 10, ztail, 0)

        def issue(q, sb, rows, sem):
            for g in range(WK // LN):
                w = wlw_v[pl.ds(q * WK + g * LN, LN)]
                sb[pl.ds(g * LN, LN)] = w & 16383
            return pltpu.async_copy(hx_hbm.at[sb], rows, sem)

        iota16 = lax.iota(jnp.int32, LN)
        cidxs = [v * LN + iota16 for v in range(NV)]

        def accum(q, db, rows):
            for kk in range(WK):
                e16 = jnp.full((LN,), q * WK + kk, jnp.int32)
                wv = plsc.load_gather(wlw_v, [e16])
                eb = plsc.load_gather(wle_v, [e16])
                av0 = lax.shift_right_logical(wv, 14) * CP
                for v in range(NV):
                    val = rows[kk, pl.ds(v * LN, LN)] * eb
                    plsc.addupdate_scatter(acc_v, [av0 + cidxs[v]], val)

        nch2 = (cur + 2 * WK - 1) // (2 * WK)
        d0 = issue(0, sb0, rows0, sem0)

        def pair(q2, carry):
            q0 = 2 * q2
            d1 = issue(q0 + 1, sb1, rows1, sem1)
            pltpu.make_async_copy(hx_hbm.at[sb0], rows0, sem0).wait()
            accum(q0, None, rows0)
            issue(q0 + 2, sb0, rows0, sem0)
            pltpu.make_async_copy(hx_hbm.at[sb1], rows1, sem1).wait()
            accum(q0 + 1, None, rows1)
            return carry

        lax.fori_loop(0, nch2, pair, 0)
        pltpu.make_async_copy(hx_hbm.at[sb0], rows0, sem0).wait()

        pltpu.sync_copy(acc_v, u_hbm.at[cid, sid])

    return k(hx, ex, src_p, dst_p)


def kernel(x, edge_index, W1, a_src1, a_dst1, b1, W2, a_src2, a_dst2, b2):
    N = x.shape[0]
    E = edge_index.shape[1]
    C1, P1 = W1.shape[1], 16
    C2, P2 = W2.shape[1], 16

    gran = NC * NS * EXCH
    EP = ((E + gran - 1) // gran) * gran
    pad = EP - E
    src_p = jnp.concatenate([edge_index[0], jnp.zeros((pad,), jnp.int32)])
    dst_p = jnp.concatenate([edge_index[1], -jnp.ones((pad,), jnp.int32)])

    hx1, aa1 = _tc_pre(x, W1, a_src1, a_dst1, C1, P1)
    ex1 = _sc_ex(aa1.reshape(-1), src_p, dst_p, N=N, E=E)
    U1 = _sc_acc(hx1, ex1, src_p, dst_p, CP=C1 + P1)
    U1 = U1.reshape(NC, NS * RPT, C1 + P1)[:, :N]
    hx2, aa2 = _tc_mid(U1, b1, W2, a_src2, a_dst2, C1, P1, C2, P2)
    ex2 = _sc_ex(aa2.reshape(-1), src_p, dst_p, N=N, E=E)
    U2 = _sc_acc(hx2, ex2, src_p, dst_p, CP=C2 + P2)
    U2 = U2.reshape(NC, NS * RPT, C2 + P2)[:, :N]
    return _tc_post(U2, b2, C2, P2)


# speedup vs baseline: 1.1917x; 1.1917x over previous
import functools

import jax
import jax.numpy as jnp
from jax import lax
from jax.experimental import pallas as pl
from jax.experimental.pallas import tpu as pltpu
from jax.experimental.pallas import tpu_sc as plsc

NC = 2
NS = 16
LN = 16
RPT = 640
EXCH = 512
SCH = 640
WK = 32
WCAP = 11264
WSZ = WCAP + 224


def _tc_pre(x, W, a_s, a_d, C, P):
    N = x.shape[0]
    F = x.shape[1]
    BN = 1000

    def body(x_ref, w_ref, as_ref, ad_ref, hx_ref, aa_ref):
        h = jnp.dot(x_ref[...], w_ref[...], preferred_element_type=jnp.float32)
        hx_ref[:, :C] = h
        col = lax.broadcasted_iota(jnp.int32, (BN, P), 1)
        hx_ref[:, C:] = jnp.where(col == 0, 1.0, 0.0)
        a1 = jnp.sum(h * as_ref[...], axis=1)
        a2 = jnp.sum(h * ad_ref[...], axis=1)
        aa_ref[...] = jnp.stack([a1, a2], axis=1)

    return pl.pallas_call(
        body,
        grid=(N // BN,),
        in_specs=[
            pl.BlockSpec((BN, F), lambda i: (i, 0)),
            pl.BlockSpec((F, C), lambda i: (0, 0)),
            pl.BlockSpec((1, C), lambda i: (0, 0)),
            pl.BlockSpec((1, C), lambda i: (0, 0)),
        ],
        out_specs=[
            pl.BlockSpec((BN, C + P), lambda i: (i, 0)),
            pl.BlockSpec((BN, 2), lambda i: (i, 0)),
        ],
        out_shape=[
            jax.ShapeDtypeStruct((N, C + P), jnp.float32),
            jax.ShapeDtypeStruct((N, 2), jnp.float32),
        ],
    )(x, W, a_s.reshape(1, C), a_d.reshape(1, C))


def _tc_mid(U, b, W, a_s, a_d, C, P, C2, P2):
    N = U.shape[1]
    BN = 1000

    def body(u_ref, b_ref, w_ref, as_ref, ad_ref, hx_ref, aa_ref):
        u = u_ref[0] + u_ref[1]
        o = u[:, :C] / (u[:, C:C + 1] + 1e-16) + b_ref[...]
        x2 = jnp.where(o > 0, o, jnp.exp(jnp.minimum(o, 0.0)) - 1.0)
        h = jnp.dot(x2, w_ref[...], preferred_element_type=jnp.float32)
        hx_ref[:, :C2] = h
        col = lax.broadcasted_iota(jnp.int32, (BN, P2), 1)
        hx_ref[:, C2:] = jnp.where(col == 0, 1.0, 0.0)
        a1 = jnp.sum(h * as_ref[...], axis=1)
        a2 = jnp.sum(h * ad_ref[...], axis=1)
        aa_ref[...] = jnp.stack([a1, a2], axis=1)

    return pl.pallas_call(
        body,
        grid=(N // BN,),
        in_specs=[
            pl.BlockSpec((2, BN, C + P), lambda i: (0, i, 0)),
            pl.BlockSpec((1, C), lambda i: (0, 0)),
            pl.BlockSpec((C, C2), lambda i: (0, 0)),
            pl.BlockSpec((1, C2), lambda i: (0, 0)),
            pl.BlockSpec((1, C2), lambda i: (0, 0)),
        ],
        out_specs=[
            pl.BlockSpec((BN, C2 + P2), lambda i: (i, 0)),
            pl.BlockSpec((BN, 2), lambda i: (i, 0)),
        ],
        out_shape=[
            jax.ShapeDtypeStruct((N, C2 + P2), jnp.float32),
            jax.ShapeDtypeStruct((N, 2), jnp.float32),
        ],
    )(U, b.reshape(1, C), W, a_s.reshape(1, C2), a_d.reshape(1, C2))


def _tc_post(U, b, C, P):
    N = U.shape[1]
    BN = 1000

    def body(u_ref, b_ref, out_ref):
        u = u_ref[0] + u_ref[1]
        o = u[:, :C] / (u[:, C:C + 1] + 1e-16) + b_ref[...]
        m = jnp.max(o, axis=1, keepdims=True)
        s = o - m
        lse = jnp.log(jnp.sum(jnp.exp(s), axis=1, keepdims=True))
        out_ref[...] = s - lse

    return pl.pallas_call(
        body,
        grid=(N // BN,),
        in_specs=[
            pl.BlockSpec((2, BN, C + P), lambda i: (0, i, 0)),
            pl.BlockSpec((1, C), lambda i: (0, 0)),
        ],
        out_specs=pl.BlockSpec((BN, C), lambda i: (i, 0)),
        out_shape=jax.ShapeDtypeStruct((N, C), jnp.float32),
    )(U, b.reshape(1, C))


_GDN = lax.GatherDimensionNumbers(
    offset_dims=(), collapsed_slice_dims=(0,), start_index_map=(0,))


def _lane_bcast(vec, idx16):
    return lax.gather(vec, idx16[:, None], _GDN, (1,),
                      mode=lax.GatherScatterMode.PROMISE_IN_BOUNDS)


_SC_PARAMS = pltpu.CompilerParams(
    needs_layout_passes=False, use_tc_tiling_on_sc=False)


@functools.partial(jax.jit, static_argnames=("N", "E"))
def _sc_ex(aa, src_p, dst_p, N, E):
    EP = src_p.shape[0]
    EPT = EP // (NC * NS)
    mesh = plsc.VectorSubcoreMesh(core_axis_name="c", subcore_axis_name="s")

    @functools.partial(
        pl.kernel,
        out_type=jax.ShapeDtypeStruct((EP,), jnp.float32),
        mesh=mesh,
        compiler_params=_SC_PARAMS,
        scratch_types=[
            pltpu.VMEM((2 * N,), jnp.float32),
            pltpu.VMEM((EXCH,), jnp.int32),
            pltpu.VMEM((EXCH,), jnp.int32),
            pltpu.VMEM((EXCH,), jnp.float32),
        ],
    )
    def k(aa_hbm, src_hbm, dst_hbm, ex_hbm, aa_v, ss_v, sd_v, se_v):
        cid = lax.axis_index("c")
        sid = lax.axis_index("s")
        wid = cid * NS + sid
        pltpu.sync_copy(aa_hbm, aa_v)
        base = wid * EPT

        def chunk(c, carry):
            off = base + c * EXCH
            pltpu.sync_copy(src_hbm.at[pl.ds(off, EXCH)], ss_v)
            pltpu.sync_copy(dst_hbm.at[pl.ds(off, EXCH)], sd_v)
            for j in range(EXCH // LN):
                s16 = ss_v[pl.ds(j * LN, LN)]
                d16 = sd_v[pl.ds(j * LN, LN)]
                d16c = jnp.maximum(d16, 0)
                z = (plsc.load_gather(aa_v, [2 * s16])
                     + plsc.load_gather(aa_v, [2 * d16c + 1]))
                e = jnp.maximum(z, 0.2 * z)
                eid = off + j * LN + lax.iota(jnp.int32, LN)
                se_v[pl.ds(j * LN, LN)] = jnp.where(eid < E, jnp.exp(e), 0.0)
            pltpu.sync_copy(se_v, ex_hbm.at[pl.ds(off, EXCH)])
            return carry

        lax.fori_loop(0, EPT // EXCH, chunk, 0)

    return k(aa, src_p, dst_p)


@functools.partial(jax.jit, static_argnames=("CP",))
def _sc_acc(hx, ex, src_p, dst_p, CP):
    EP = src_p.shape[0]
    EH = EP // NC
    NV = CP // LN
    mesh = plsc.VectorSubcoreMesh(core_axis_name="c", subcore_axis_name="s")

    @functools.partial(
        pl.kernel,
        out_type=jax.ShapeDtypeStruct((NC, NS, RPT * CP), jnp.float32),
        mesh=mesh,
        compiler_params=_SC_PARAMS,
        scratch_types=[
            pltpu.VMEM((WSZ,), jnp.int32),
            pltpu.VMEM((WSZ,), jnp.float32),
            pltpu.VMEM((SCH,), jnp.int32),
            pltpu.VMEM((SCH,), jnp.int32),
            pltpu.VMEM((SCH,), jnp.float32),
            pltpu.VMEM((SCH,), jnp.int32),
            pltpu.VMEM((SCH,), jnp.int32),
            pltpu.VMEM((SCH,), jnp.float32),
            pltpu.VMEM((WK, CP), jnp.float32),
            pltpu.VMEM((WK, CP), jnp.float32),
            pltpu.VMEM((WK,), jnp.int32),
            pltpu.VMEM((WK,), jnp.int32),
            pltpu.VMEM((RPT * CP,), jnp.float32),
            pltpu.SemaphoreType.DMA,
            pltpu.SemaphoreType.DMA,
            pltpu.SemaphoreType.DMA,
            pltpu.SemaphoreType.DMA,
        ],
    )
    def k(hx_hbm, ex_hbm, src_hbm, dst_hbm, u_hbm,
          wlw_v, wle_v, ss0, sd0, se0, ss1, sd1, se1,
          rows0, rows1, sb0, sb1, acc_v, sem0, sem1, sems, semt):
        cid = lax.axis_index("c")
        sid = lax.axis_index("s")
        mybase = sid * RPT
        hb = cid * EH

        zero16 = jnp.zeros((LN,), jnp.float32)

        def zr(g, carry):
            for v in range(NV):
                acc_v[pl.ds(g * CP + v * LN, LN)] = zero16
            return carry

        lax.fori_loop(0, RPT, zr, 0)

        def scan_one(ss, sd, se, cur):
            for j in range(SCH // LN):
                s16 = ss[pl.ds(j * LN, LN)]
                d16 = sd[pl.ds(j * LN, LN)]
                e16 = se[pl.ds(j * LN, LN)]
                dl = d16 - mybase
                m = (dl >= 0) & (dl < RPT)
                w = s16 | (jnp.maximum(dl, 0) << 14)
                plsc.store_compressed(wlw_v.at[pl.ds(cur, LN)], w, mask=m)
                plsc.store_compressed(wle_v.at[pl.ds(cur, LN)], e16, mask=m)
                cur = cur + plsc.all_reduce_population_count(m)[0]
            return jnp.minimum(cur, WCAP)

        NSC = EH // SCH

        def issue_scan(c, ss, sd, se, sem):
            off = hb + jnp.minimum(c, NSC - 1) * SCH
            pltpu.async_copy(src_hbm.at[pl.ds(off, SCH)], ss, sem)
            pltpu.async_copy(dst_hbm.at[pl.ds(off, SCH)], sd, sem)
            pltpu.async_copy(ex_hbm.at[pl.ds(off, SCH)], se, sem)

        def wait_scan(ss, sd, se, sem):
            dummy = src_hbm.at[pl.ds(0, SCH)]
            pltpu.make_async_copy(dummy, ss, sem).wait()
            pltpu.make_async_copy(dummy, sd, sem).wait()
            pltpu.make_async_copy(ex_hbm.at[pl.ds(0, SCH)], se, sem).wait()

        issue_scan(0, ss0, sd0, se0, sems)

        def scan_pair(c2, cur):
            c0 = 2 * c2
            issue_scan(c0 + 1, ss1, sd1, se1, semt)
            wait_scan(ss0, sd0, se0, sems)
            cur = scan_one(ss0, sd0, se0, cur)
            issue_scan(c0 + 2, ss0, sd0, se0, sems)
            wait_scan(ss1, sd1, se1, semt)
            return scan_one(ss1, sd1, se1, cur)

        cur = lax.fori_loop(0, NSC // 2, scan_pair, 0)
        wait_scan(ss0, sd0, se0, sems)

        def ztail(t, carry):
            wlw_v[pl.ds(cur + t * LN, LN)] = jnp.zeros((LN,), jnp.int32)
            wle_v[pl.ds(cur + t * LN, LN)] = zero16
            return carry

        lax.fori_loop(0, 10, ztail, 0)

        def issue(q, sb, rows, sem):
            for g in range(WK // LN):
                w = wlw_v[pl.ds(q * WK + g * LN, LN)]
                sb[pl.ds(g * LN, LN)] = w & 16383
            return pltpu.async_copy(hx_hbm.at[sb], rows, sem)

        iota16 = lax.iota(jnp.int32, LN)
        cidxs = [v * LN + iota16 for v in range(NV)]

        lane_splats = [jnp.full((LN,), u, jnp.int32) for u in range(LN)]

        def accum(q, db, rows):
            for g in range(WK // LN):
                w16 = wlw_v[pl.ds(q * WK + g * LN, LN)]
                x16 = wle_v[pl.ds(q * WK + g * LN, LN)]
                a16 = lax.shift_right_logical(w16, 14) * CP
                for u in range(LN):
                    av0 = _lane_bcast(a16, lane_splats[u])
                    eb = _lane_bcast(x16, lane_splats[u])
                    kk = g * LN + u
                    for v in range(NV):
                        val = rows[kk, pl.ds(v * LN, LN)] * eb
                        plsc.addupdate_scatter(acc_v, [av0 + cidxs[v]], val)

        nch2 = (cur + 2 * WK - 1) // (2 * WK)
        d0 = issue(0, sb0, rows0, sem0)

        def pair(q2, carry):
            q0 = 2 * q2
            d1 = issue(q0 + 1, sb1, rows1, sem1)
            pltpu.make_async_copy(hx_hbm.at[sb0], rows0, sem0).wait()
            accum(q0, None, rows0)
            issue(q0 + 2, sb0, rows0, sem0)
            pltpu.make_async_copy(hx_hbm.at[sb1], rows1, sem1).wait()
            accum(q0 + 1, None, rows1)
            return carry

        lax.fori_loop(0, nch2, pair, 0)
        pltpu.make_async_copy(hx_hbm.at[sb0], rows0, sem0).wait()

        pltpu.sync_copy(acc_v, u_hbm.at[cid, sid])

    return k(hx, ex, src_p, dst_p)


def kernel(x, edge_index, W1, a_src1, a_dst1, b1, W2, a_src2, a_dst2, b2):
    N = x.shape[0]
    E = edge_index.shape[1]
    C1, P1 = W1.shape[1], 16
    C2, P2 = W2.shape[1], 16

    gran = NC * NS * EXCH
    EP = ((E + gran - 1) // gran) * gran
    pad = EP - E
    src_p = jnp.concatenate([edge_index[0], jnp.zeros((pad,), jnp.int32)])
    dst_p = jnp.concatenate([edge_index[1], -jnp.ones((pad,), jnp.int32)])

    hx1, aa1 = _tc_pre(x, W1, a_src1, a_dst1, C1, P1)
    ex1 = _sc_ex(aa1.reshape(-1), src_p, dst_p, N=N, E=E)
    U1 = _sc_acc(hx1, ex1, src_p, dst_p, CP=C1 + P1)
    U1 = U1.reshape(NC, NS * RPT, C1 + P1)[:, :N]
    hx2, aa2 = _tc_mid(U1, b1, W2, a_src2, a_dst2, C1, P1, C2, P2)
    ex2 = _sc_ex(aa2.reshape(-1), src_p, dst_p, N=N, E=E)
    U2 = _sc_acc(hx2, ex2, src_p, dst_p, CP=C2 + P2)
    U2 = U2.reshape(NC, NS * RPT, C2 + P2)[:, :N]
    return _tc_post(U2, b2, C2, P2)
